# baseline (speedup 1.0000x reference)
import functools

import jax
import jax.numpy as jnp
from jax import lax
from jax.experimental import pallas as pl
from jax.experimental.pallas import tpu as pltpu
from jax.experimental.pallas import tpu_sc as plsc

B, C, N_IN = 2, 32, 786432
K_OUT = N_IN // 4
M_IN = B * C * N_IN
M_OUT = M_IN // 4

NC, NS = 2, 16
NW = NC * NS
IN_PER_W = M_IN // NW
OUT_PER_W = IN_PER_W // 4

IN_CHUNK = 32768
OUT_CHUNK = IN_CHUNK // 4
N_ITERS = IN_PER_W // IN_CHUNK
UNROLL = 8
STEPS = IN_CHUNK // (16 * UNROLL)

_mesh = plsc.VectorSubcoreMesh(core_axis_name="c", subcore_axis_name="s")


@functools.partial(
    pl.kernel,
    out_type=jax.ShapeDtypeStruct((M_OUT,), jnp.float32),
    mesh=_mesh,
    scratch_types=[
        pltpu.VMEM((IN_CHUNK,), jnp.float32),
        pltpu.VMEM((IN_CHUNK,), jnp.float32),
        pltpu.VMEM((OUT_CHUNK,), jnp.float32),
        pltpu.VMEM((OUT_CHUNK,), jnp.float32),
        pltpu.SemaphoreType.DMA,
        pltpu.SemaphoreType.DMA,
        pltpu.SemaphoreType.DMA,
        pltpu.SemaphoreType.DMA,
    ],
    compiler_params=pltpu.CompilerParams(needs_layout_passes=False),
)
def _sc_pool4(x_hbm, out_hbm, in_v0, in_v1, out_v0, out_v1, si0, si1, so0, so1):
    wid = lax.axis_index("s") * NC + lax.axis_index("c")
    base_in = wid * IN_PER_W
    base_out = wid * OUT_PER_W
    in_bufs = (in_v0, in_v1)
    out_bufs = (out_v0, out_v1)
    in_sems = (si0, si1)
    out_sems = (so0, so1)

    def in_copy(i, slot):
        return pltpu.make_async_copy(
            x_hbm.at[pl.ds(base_in + i * IN_CHUNK, IN_CHUNK)],
            in_bufs[slot],
            in_sems[slot],
        )

    def out_copy(i, slot):
        return pltpu.make_async_copy(
            out_bufs[slot],
            out_hbm.at[pl.ds(base_out + i * OUT_CHUNK, OUT_CHUNK)],
            out_sems[slot],
        )

    iota = lax.iota(jnp.int32, 16)
    perm1 = iota ^ 1
    perm2 = iota ^ 2
    omask = (iota & 3) == 0
    oadd = iota >> 2

    in_copy(0, 0).start()
    in_copy(1, 1).start()

    def one_iter(i, slot):
        in_copy(i, slot).wait()

        @pl.when(i >= 2)
        def _():
            out_copy(i - 2, slot).wait()

        in_ref = in_bufs[slot]
        out_ref = out_bufs[slot]

        def step(t, idx_base):
            b = t * (16 * UNROLL)
            for u in range(UNROLL):
                v = in_ref[pl.ds(b + u * 16, 16)]
                m1 = jnp.maximum(v, jnp.take_along_axis(v, perm1, axis=0))
                m2 = jnp.maximum(m1, jnp.take_along_axis(m1, perm2, axis=0))
                plsc.store_scatter(out_ref, [idx_base + 4 * u], m2, mask=omask)
            return idx_base + 4 * UNROLL

        lax.fori_loop(0, STEPS, step, oadd, unroll=False)

        out_copy(i, slot).start()

        @pl.when(i + 2 < N_ITERS)
        def _():
            in_copy(i + 2, slot).start()

    def pair(g, carry):
        one_iter(2 * g, 0)
        one_iter(2 * g + 1, 1)
        return carry

    lax.fori_loop(0, N_ITERS // 2, pair, 0, unroll=False)

    out_copy(N_ITERS - 2, 0).wait()
    out_copy(N_ITERS - 1, 1).wait()


def kernel(x, children_idx, cell_ids_out):
    del children_idx
    y_flat = _sc_pool4(x.reshape(M_IN))
    return y_flat.reshape(B, C, K_OUT), cell_ids_out


# speedup vs baseline: 1.6331x; 1.6331x over previous
import functools

import jax
import jax.numpy as jnp
from jax import lax
from jax.experimental import pallas as pl
from jax.experimental.pallas import tpu as pltpu
from jax.experimental.pallas import tpu_sc as plsc

B, C, N_IN = 2, 32, 786432
K_OUT = N_IN // 4
M_IN = B * C * N_IN
M_OUT = M_IN // 4

NC, NS = 2, 16
NW = NC * NS
IN_PER_W = M_IN // NW
OUT_PER_W = IN_PER_W // 4

IN_CHUNK = 32768
OUT_CHUNK = IN_CHUNK // 4
N_ITERS = IN_PER_W // IN_CHUNK
UNROLL = 8
STEPS = IN_CHUNK // (32 * UNROLL)

_mesh = plsc.VectorSubcoreMesh(core_axis_name="c", subcore_axis_name="s")


@functools.partial(
    pl.kernel,
    out_type=jax.ShapeDtypeStruct((M_OUT,), jnp.float32),
    mesh=_mesh,
    scratch_types=[
        pltpu.VMEM((IN_CHUNK,), jnp.float32),
        pltpu.VMEM((IN_CHUNK,), jnp.float32),
        pltpu.VMEM((OUT_CHUNK,), jnp.float32),
        pltpu.VMEM((OUT_CHUNK,), jnp.float32),
        pltpu.SemaphoreType.DMA,
        pltpu.SemaphoreType.DMA,
        pltpu.SemaphoreType.DMA,
        pltpu.SemaphoreType.DMA,
    ],
    compiler_params=pltpu.CompilerParams(needs_layout_passes=False),
)
def _sc_pool4(x_hbm, out_hbm, in_v0, in_v1, out_v0, out_v1, si0, si1, so0, so1):
    wid = lax.axis_index("s") * NC + lax.axis_index("c")
    base_in = wid * IN_PER_W
    base_out = wid * OUT_PER_W
    in_bufs = (in_v0, in_v1)
    out_bufs = (out_v0, out_v1)
    in_sems = (si0, si1)
    out_sems = (so0, so1)

    def in_copy(i, slot):
        return pltpu.make_async_copy(
            x_hbm.at[pl.ds(base_in + i * IN_CHUNK, IN_CHUNK)],
            in_bufs[slot],
            in_sems[slot],
        )

    def out_copy(i, slot):
        return pltpu.make_async_copy(
            out_bufs[slot],
            out_hbm.at[pl.ds(base_out + i * OUT_CHUNK, OUT_CHUNK)],
            out_sems[slot],
        )

    iota = lax.iota(jnp.int32, 16)
    idx_even = iota * 2
    perm1 = iota ^ 1
    omask = (iota & 1) == 0
    oadd = iota >> 1

    in_copy(0, 0).start()
    in_copy(1, 1).start()

    def one_iter(i, slot):
        in_copy(i, slot).wait()

        @pl.when(i >= 2)
        def _():
            out_copy(i - 2, slot).wait()

        in_ref = in_bufs[slot]
        out_ref = out_bufs[slot]

        def step(t, carry):
            idx_base, oidx_base = carry
            for u in range(UNROLL):
                ia = idx_base + 32 * u
                ga = plsc.load_gather(in_ref, [ia])
                gb = plsc.load_gather(in_ref, [ia + 1])
                p = jnp.maximum(ga, gb)
                m = jnp.maximum(p, jnp.take_along_axis(p, perm1, axis=0))
                plsc.store_scatter(out_ref, [oidx_base + 8 * u], m, mask=omask)
            return (idx_base + 32 * UNROLL, oidx_base + 8 * UNROLL)

        lax.fori_loop(0, STEPS, step, (idx_even, oadd), unroll=False)

        out_copy(i, slot).start()

        @pl.when(i + 2 < N_ITERS)
        def _():
            in_copy(i + 2, slot).start()

    def pair(g, carry):
        one_iter(2 * g, 0)
        one_iter(2 * g + 1, 1)
        return carry

    lax.fori_loop(0, N_ITERS // 2, pair, 0, unroll=False)

    out_copy(N_ITERS - 2, 0).wait()
    out_copy(N_ITERS - 1, 1).wait()


def kernel(x, children_idx, cell_ids_out):
    del children_idx
    y_flat = _sc_pool4(x.reshape(M_IN))
    return y_flat.reshape(B, C, K_OUT), cell_ids_out


# speedup vs baseline: 4.5902x; 2.8108x over previous
import functools

import jax
import jax.numpy as jnp
from jax import lax
from jax.experimental import pallas as pl
from jax.experimental.pallas import tpu as pltpu
from jax.experimental.pallas import tpu_sc as plsc

B, C, N_IN = 2, 32, 786432
K_OUT = N_IN // 4

NC, NS = 2, 16

IN_CHUNK = 32768
OUT_CHUNK = IN_CHUNK // 4
N_ITERS = N_IN // IN_CHUNK
UNROLL = 4
STEPS = OUT_CHUNK // (16 * UNROLL)

_mesh = plsc.VectorSubcoreMesh(core_axis_name="c", subcore_axis_name="s")


@functools.partial(
    pl.kernel,
    out_type=jax.ShapeDtypeStruct((B, C, K_OUT), jnp.float32),
    mesh=_mesh,
    scratch_types=[
        pltpu.VMEM((IN_CHUNK,), jnp.float32),
        pltpu.VMEM((IN_CHUNK,), jnp.float32),
        pltpu.VMEM((OUT_CHUNK,), jnp.float32),
        pltpu.VMEM((OUT_CHUNK,), jnp.float32),
        pltpu.SemaphoreType.DMA,
        pltpu.SemaphoreType.DMA,
        pltpu.SemaphoreType.DMA,
        pltpu.SemaphoreType.DMA,
    ],
    compiler_params=pltpu.CompilerParams(needs_layout_passes=False),
)
def _sc_pool4(x_hbm, out_hbm, in_v0, in_v1, out_v0, out_v1, si0, si1, so0, so1):
    w = lax.axis_index("s") * NC + lax.axis_index("c")
    in_bufs = (in_v0, in_v1)
    out_bufs = (out_v0, out_v1)
    in_sems = (si0, si1)
    out_sems = (so0, so1)

    iota4 = lax.iota(jnp.int32, 16) * 4

    for b in range(B):
        def in_copy(i, slot, b=b):
            return pltpu.make_async_copy(
                x_hbm.at[b, w, pl.ds(i * IN_CHUNK, IN_CHUNK)],
                in_bufs[slot],
                in_sems[slot],
            )

        def out_copy(i, slot, b=b):
            return pltpu.make_async_copy(
                out_bufs[slot],
                out_hbm.at[b, w, pl.ds(i * OUT_CHUNK, OUT_CHUNK)],
                out_sems[slot],
            )

        in_copy(0, 0).start()
        in_copy(1, 1).start()

        def one_iter(i, slot):
            in_copy(i, slot).wait()

            @pl.when(i >= 2)
            def _():
                out_copy(i - 2, slot).wait()

            in_ref = in_bufs[slot]
            out_ref = out_bufs[slot]

            def step(t, carry):
                for u in range(UNROLL):
                    o = t * (16 * UNROLL) + u * 16
                    idx = iota4 + o * 4
                    g0 = plsc.load_gather(in_ref, [idx])
                    g1 = plsc.load_gather(in_ref, [idx + 1])
                    g2 = plsc.load_gather(in_ref, [idx + 2])
                    g3 = plsc.load_gather(in_ref, [idx + 3])
                    out_ref[pl.ds(o, 16)] = jnp.maximum(
                        jnp.maximum(g0, g1), jnp.maximum(g2, g3)
                    )
                return carry

            lax.fori_loop(0, STEPS, step, 0, unroll=False)

            out_copy(i, slot).start()

            @pl.when(i + 2 < N_ITERS)
            def _():
                in_copy(i + 2, slot).start()

        def pair(g, carry):
            one_iter(2 * g, 0)
            one_iter(2 * g + 1, 1)
            return carry

        lax.fori_loop(0, N_ITERS // 2, pair, 0, unroll=False)

        out_copy(N_ITERS - 2, 0).wait()
        out_copy(N_ITERS - 1, 1).wait()


def kernel(x, children_idx, cell_ids_out):
    del children_idx
    return _sc_pool4(x), cell_ids_out
